# baseline (speedup 1.0000x reference)
import functools

import jax
import jax.numpy as jnp
from jax import lax
from jax.experimental import pallas as pl
from jax.experimental.pallas import tpu as pltpu
from jax.experimental.pallas import tpu_sc as plsc

_N_ENT = 50000
_N_USERS = 1024
_E = 800000
_C = 64
_N_REL = 16
_HOPS = 2

_HALF = _C // 2
_CHUNK = 128
_NSUB = 16
_GRP = 4
_NCHUNKP = 6272
_EPAD = _NCHUNKP * _CHUNK
_TCH = _NCHUNKP // _NSUB
_NGRP = _TCH // _GRP
_WCHUNK = 400
_NWCHUNK = _N_ENT // _WCHUNK

_LANES = 16

_mesh = plsc.VectorSubcoreMesh(core_axis_name="c", subcore_axis_name="s")


@functools.partial(
    pl.kernel,
    out_type=jax.ShapeDtypeStruct((2, _N_ENT, _HALF), jnp.float32),
    mesh=_mesh,
    scratch_types=[
        pltpu.VMEM((_GRP, 4, _CHUNK), jnp.int32),
        pltpu.VMEM((_GRP, _CHUNK, _HALF), jnp.float32),
        pltpu.VMEM((_N_REL, _HALF), jnp.float32),
        pltpu.VMEM_SHARED((_N_ENT, _HALF), jnp.float32),
        pltpu.SemaphoreType.DMA((_GRP,)),
        pltpu.SemaphoreType.DMA((_GRP,)),
        pltpu.SemaphoreType.DMA((_GRP,)),
    ],
    compiler_params=pltpu.CompilerParams(use_tc_tiling_on_sc=False,
                                         needs_layout_passes=False),
)
def _sc_aggregate(ent_hbm, edges_hbm, relw_hbm, zeros_hbm, out_hbm,
                  ebuf, rows, relw, acc, sem_e, sem_g, sem_s):
    c = lax.axis_index("c")
    s = lax.axis_index("s")

    pltpu.sync_copy(relw_hbm.at[c], relw)

    @pl.loop(s, _NWCHUNK, step=_NSUB)
    def _zcopy(k):
        pltpu.sync_copy(zeros_hbm, acc.at[pl.ds(k * _WCHUNK, _WCHUNK)])

    plsc.subcore_barrier()

    entc = ent_hbm.at[c]

    @pl.loop(0, _NGRP)
    def _group(gi):
        jbase = s + gi * (_GRP * _NSUB)

        descs_e = []
        for g in range(_GRP):
            descs_e.append(pltpu.async_copy(
                edges_hbm.at[jbase + g * _NSUB], ebuf.at[g], sem_e.at[g]))

        descs_g = []
        for g in range(_GRP):
            descs_e[g].wait()
            descs_g.append(pltpu.async_copy(
                entc.at[ebuf.at[g, 0]], rows.at[g], sem_g.at[g]))

        descs_s = []
        for g in range(_GRP):
            descs_g[g].wait()

            @plsc.parallel_loop(0, _CHUNK // _LANES, unroll=2)
            def _sub(q):
                tvec = ebuf[g, 2, pl.ds(q * _LANES, _LANES)]
                uvec = plsc.bitcast(ebuf[g, 3, pl.ds(q * _LANES, _LANES)],
                                    jnp.float32)
                for ii in range(_LANES):
                    i = q * _LANES + ii
                    t = tvec[ii]
                    u = uvec[ii]
                    r0 = relw[t, pl.ds(0, _LANES)] * u
                    r1 = relw[t, pl.ds(_LANES, _LANES)] * u
                    rows[g, i, pl.ds(0, _LANES)] = (
                        rows[g, i, pl.ds(0, _LANES)] * r0)
                    rows[g, i, pl.ds(_LANES, _LANES)] = (
                        rows[g, i, pl.ds(_LANES, _LANES)] * r1)

            descs_s.append(pltpu.async_copy(
                rows.at[g], acc.at[ebuf.at[g, 1]], sem_s.at[g], add=True))

        for g in range(_GRP):
            descs_s[g].wait()

    plsc.subcore_barrier()

    outc = out_hbm.at[c]

    @pl.loop(s, _NWCHUNK, step=_NSUB)
    def _wb(k):
        off = k * _WCHUNK
        pltpu.sync_copy(acc.at[pl.ds(off, _WCHUNK)],
                        outc.at[pl.ds(off, _WCHUNK)])


_UBLK = 64
_USTEPS = _N_USERS // _UBLK


def _mm_body(im_ref, ent_ref, resin_ref, out_ref):
    a = jnp.dot(im_ref[...], ent_ref[...], preferred_element_type=jnp.float32)
    n = jnp.sqrt(jnp.sum(a * a, axis=1, keepdims=True))
    out_ref[...] = resin_ref[...] + a / jnp.maximum(n, 1e-12)


_user_hop = pl.pallas_call(
    _mm_body,
    grid=(_USTEPS,),
    in_specs=[
        pl.BlockSpec((_UBLK, _N_ENT), lambda u: (u, 0)),
        pl.BlockSpec((_N_ENT, _C), lambda u: (0, 0)),
        pl.BlockSpec((_UBLK, _C), lambda u: (u, 0)),
    ],
    out_specs=pl.BlockSpec((_UBLK, _C), lambda u: (u, 0)),
    out_shape=jax.ShapeDtypeStruct((_N_USERS, _C), jnp.float32),
)


_RBLK = 2000


def _norm_body(agg_ref, resin_ref, resout_ref, full_ref, split_ref):
    h0 = agg_ref[0]
    h1 = agg_ref[1]
    n2 = (jnp.sum(h0 * h0, axis=1, keepdims=True)
          + jnp.sum(h1 * h1, axis=1, keepdims=True))
    inv = 1.0 / jnp.maximum(jnp.sqrt(n2), 1e-12)
    e0 = h0 * inv
    e1 = h1 * inv
    full = jnp.concatenate([e0, e1], axis=1)
    resout_ref[...] = resin_ref[...] + full
    full_ref[...] = full
    split_ref[0] = e0
    split_ref[1] = e1


_ent_norm = pl.pallas_call(
    _norm_body,
    grid=(_N_ENT // _RBLK,),
    in_specs=[
        pl.BlockSpec((2, _RBLK, _HALF), lambda i: (0, i, 0)),
        pl.BlockSpec((_RBLK, _C), lambda i: (i, 0)),
    ],
    out_specs=[
        pl.BlockSpec((_RBLK, _C), lambda i: (i, 0)),
        pl.BlockSpec((_RBLK, _C), lambda i: (i, 0)),
        pl.BlockSpec((2, _RBLK, _HALF), lambda i: (0, i, 0)),
    ],
    out_shape=[
        jax.ShapeDtypeStruct((_N_ENT, _C), jnp.float32),
        jax.ShapeDtypeStruct((_N_ENT, _C), jnp.float32),
        jax.ShapeDtypeStruct((2, _N_ENT, _HALF), jnp.float32),
    ],
)


def kernel(user_emb, entity_emb, entity_2nd_emb, user_2nd_emb,
           edge_index, edge_type, interact_mat, weight, triplet_mask):
    del entity_2nd_emb, user_2nd_emb

    ei = edge_index.astype(jnp.int32)
    pad = _EPAD - _E
    tail = jnp.pad(ei[1], (0, pad)).reshape(_NCHUNKP, _CHUNK)
    head = jnp.pad(ei[0], (0, pad)).reshape(_NCHUNKP, _CHUNK)
    et = jnp.pad(edge_type.astype(jnp.int32), (0, pad)).reshape(_NCHUNKP,
                                                                _CHUNK)
    umbits = lax.bitcast_convert_type(
        jnp.pad(triplet_mask, (0, pad)), jnp.int32).reshape(_NCHUNKP, _CHUNK)
    edges_packed = jnp.stack([tail, head, et, umbits], axis=1)

    w2 = jnp.roll(weight, 1, axis=0)
    relw_split = jnp.stack([w2[:, :_HALF], w2[:, _HALF:]], axis=0)
    zblock = jnp.zeros((_WCHUNK, _HALF), jnp.float32)

    ent_full = entity_emb
    ent_split = jnp.stack([entity_emb[:, :_HALF], entity_emb[:, _HALF:]],
                          axis=0)
    ent_res = entity_emb
    usr_res = user_emb

    for _ in range(_HOPS):
        agg = _sc_aggregate(ent_split, edges_packed, relw_split, zblock)
        usr_res = _user_hop(interact_mat, ent_full, usr_res)
        ent_res, ent_full, ent_split = _ent_norm(agg, ent_res)

    return ent_res, usr_res, triplet_mask


# speedup vs baseline: 1.1072x; 1.1072x over previous
import functools

import jax
import jax.numpy as jnp
from jax import lax
from jax.experimental import pallas as pl
from jax.experimental.pallas import tpu as pltpu
from jax.experimental.pallas import tpu_sc as plsc

_N_ENT = 50000
_N_USERS = 1024
_E = 800000
_C = 64
_N_REL = 16
_HOPS = 2

_HALF = _C // 2
_CHUNK = 128
_NSUB = 16
_GRP = 6
_NCHUNKP = 6336
_EPAD = _NCHUNKP * _CHUNK
_TCH = _NCHUNKP // _NSUB
_NBLK = _TCH // _GRP
_DE = 4
_DG = 2
_WCHUNK = 400
_NWCHUNK = _N_ENT // _WCHUNK

_LANES = 16

_mesh = plsc.VectorSubcoreMesh(core_axis_name="c", subcore_axis_name="s")


@functools.partial(
    pl.kernel,
    out_type=jax.ShapeDtypeStruct((2, _N_ENT, _HALF), jnp.float32),
    mesh=_mesh,
    scratch_types=[
        pltpu.VMEM((_GRP, 4, _CHUNK), jnp.int32),
        pltpu.VMEM((_GRP, _CHUNK, _HALF), jnp.float32),
        pltpu.VMEM((_N_REL, _HALF), jnp.float32),
        pltpu.VMEM_SHARED((_N_ENT, _HALF), jnp.float32),
        pltpu.SemaphoreType.DMA((_GRP,)),
        pltpu.SemaphoreType.DMA((_GRP,)),
        pltpu.SemaphoreType.DMA((_GRP,)),
    ],
    compiler_params=pltpu.CompilerParams(use_tc_tiling_on_sc=False,
                                         needs_layout_passes=False),
)
def _sc_aggregate(ent_hbm, edges_hbm, relw_hbm, zeros_hbm, out_hbm,
                  ebuf, rows, relw, acc, sem_e, sem_g, sem_s):
    c = lax.axis_index("c")
    s = lax.axis_index("s")

    pltpu.sync_copy(relw_hbm.at[c], relw)

    @pl.loop(s, _NWCHUNK, step=_NSUB)
    def _zcopy(k):
        pltpu.sync_copy(zeros_hbm, acc.at[pl.ds(k * _WCHUNK, _WCHUNK)])

    plsc.subcore_barrier()

    entc = ent_hbm.at[c]

    def chunk_of(k):
        return s + k * _NSUB

    def fire_e(slot, k):
        pltpu.async_copy(edges_hbm.at[chunk_of(k)], ebuf.at[slot],
                         sem_e.at[slot])

    def wait_e(slot):
        pltpu.make_async_copy(edges_hbm.at[0], ebuf.at[slot],
                              sem_e.at[slot]).wait()

    def fire_g(slot):
        pltpu.async_copy(entc.at[ebuf.at[slot, 0]], rows.at[slot],
                         sem_g.at[slot])

    def wait_g(slot):
        pltpu.make_async_copy(entc.at[ebuf.at[slot, 0]], rows.at[slot],
                              sem_g.at[slot]).wait()

    def fire_s(slot):
        pltpu.async_copy(rows.at[slot], acc.at[ebuf.at[slot, 1]],
                         sem_s.at[slot], add=True)

    def wait_s(slot):
        pltpu.make_async_copy(rows.at[slot], acc.at[ebuf.at[slot, 1]],
                              sem_s.at[slot]).wait()

    def compute(slot):
        @pl.loop(0, _CHUNK // _LANES)
        def _sub(q):
            tvec = ebuf[slot, 2, pl.ds(q * _LANES, _LANES)]
            uvec = plsc.bitcast(ebuf[slot, 3, pl.ds(q * _LANES, _LANES)],
                                jnp.float32)
            for ii in range(_LANES):
                i = q * _LANES + ii
                t = tvec[ii]
                u = uvec[ii]
                r0 = relw[t, pl.ds(0, _LANES)] * u
                r1 = relw[t, pl.ds(_LANES, _LANES)] * u
                rows[slot, i, pl.ds(0, _LANES)] = (
                    rows[slot, i, pl.ds(0, _LANES)] * r0)
                rows[slot, i, pl.ds(_LANES, _LANES)] = (
                    rows[slot, i, pl.ds(_LANES, _LANES)] * r1)

    for k0 in range(_DE):
        fire_e(k0, k0)
    for k0 in range(_DG):
        wait_e(k0)
        fire_g(k0)

    @pl.loop(0, _NBLK)
    def _block(bi):
        for q in range(_GRP):
            k = bi * _GRP + q
            wait_g(q)
            compute(q)
            fire_s(q)

            r_e = (q + _DE) % _GRP

            @pl.when(jnp.logical_and(k >= _GRP - _DE, k < _TCH - _DE))
            def _():
                wait_s(r_e)
                fire_e(r_e, k + _DE)

            @pl.when(k < _GRP - _DE)
            def _():
                fire_e(r_e, k + _DE)

            r_g = (q + _DG) % _GRP

            @pl.when(k < _TCH - _DG)
            def _():
                wait_e(r_g)
                fire_g(r_g)

    for q in range(_GRP):
        wait_s(q)

    plsc.subcore_barrier()

    outc = out_hbm.at[c]

    @pl.loop(s, _NWCHUNK, step=_NSUB)
    def _wb(k):
        off = k * _WCHUNK
        pltpu.sync_copy(acc.at[pl.ds(off, _WCHUNK)],
                        outc.at[pl.ds(off, _WCHUNK)])


_UBLK = 64
_USTEPS = _N_USERS // _UBLK


def _mm_body(im_ref, ent_ref, resin_ref, out_ref):
    a = jnp.dot(im_ref[...], ent_ref[...], preferred_element_type=jnp.float32)
    n = jnp.sqrt(jnp.sum(a * a, axis=1, keepdims=True))
    out_ref[...] = resin_ref[...] + a / jnp.maximum(n, 1e-12)


_user_hop = pl.pallas_call(
    _mm_body,
    grid=(_USTEPS,),
    in_specs=[
        pl.BlockSpec((_UBLK, _N_ENT), lambda u: (u, 0)),
        pl.BlockSpec((_N_ENT, _C), lambda u: (0, 0)),
        pl.BlockSpec((_UBLK, _C), lambda u: (u, 0)),
    ],
    out_specs=pl.BlockSpec((_UBLK, _C), lambda u: (u, 0)),
    out_shape=jax.ShapeDtypeStruct((_N_USERS, _C), jnp.float32),
)


_RBLK = 2000


def _norm_body(agg_ref, resin_ref, resout_ref, full_ref, split_ref):
    h0 = agg_ref[0]
    h1 = agg_ref[1]
    n2 = (jnp.sum(h0 * h0, axis=1, keepdims=True)
          + jnp.sum(h1 * h1, axis=1, keepdims=True))
    inv = 1.0 / jnp.maximum(jnp.sqrt(n2), 1e-12)
    e0 = h0 * inv
    e1 = h1 * inv
    full = jnp.concatenate([e0, e1], axis=1)
    resout_ref[...] = resin_ref[...] + full
    full_ref[...] = full
    split_ref[0] = e0
    split_ref[1] = e1


_ent_norm = pl.pallas_call(
    _norm_body,
    grid=(_N_ENT // _RBLK,),
    in_specs=[
        pl.BlockSpec((2, _RBLK, _HALF), lambda i: (0, i, 0)),
        pl.BlockSpec((_RBLK, _C), lambda i: (i, 0)),
    ],
    out_specs=[
        pl.BlockSpec((_RBLK, _C), lambda i: (i, 0)),
        pl.BlockSpec((_RBLK, _C), lambda i: (i, 0)),
        pl.BlockSpec((2, _RBLK, _HALF), lambda i: (0, i, 0)),
    ],
    out_shape=[
        jax.ShapeDtypeStruct((_N_ENT, _C), jnp.float32),
        jax.ShapeDtypeStruct((_N_ENT, _C), jnp.float32),
        jax.ShapeDtypeStruct((2, _N_ENT, _HALF), jnp.float32),
    ],
)


def kernel(user_emb, entity_emb, entity_2nd_emb, user_2nd_emb,
           edge_index, edge_type, interact_mat, weight, triplet_mask):
    del entity_2nd_emb, user_2nd_emb

    ei = edge_index.astype(jnp.int32)
    pad = _EPAD - _E
    tail = jnp.pad(ei[1], (0, pad)).reshape(_NCHUNKP, _CHUNK)
    head = jnp.pad(ei[0], (0, pad)).reshape(_NCHUNKP, _CHUNK)
    et = jnp.pad(edge_type.astype(jnp.int32), (0, pad)).reshape(_NCHUNKP,
                                                                _CHUNK)
    umbits = lax.bitcast_convert_type(
        jnp.pad(triplet_mask, (0, pad)), jnp.int32).reshape(_NCHUNKP, _CHUNK)
    edges_packed = jnp.stack([tail, head, et, umbits], axis=1)

    w2 = jnp.roll(weight, 1, axis=0)
    relw_split = jnp.stack([w2[:, :_HALF], w2[:, _HALF:]], axis=0)
    zblock = jnp.zeros((_WCHUNK, _HALF), jnp.float32)

    ent_full = entity_emb
    ent_split = jnp.stack([entity_emb[:, :_HALF], entity_emb[:, _HALF:]],
                          axis=0)
    ent_res = entity_emb
    usr_res = user_emb

    for _ in range(_HOPS):
        agg = _sc_aggregate(ent_split, edges_packed, relw_split, zblock)
        usr_res = _user_hop(interact_mat, ent_full, usr_res)
        ent_res, ent_full, ent_split = _ent_norm(agg, ent_res)

    return ent_res, usr_res, triplet_mask


# speedup vs baseline: 1.1339x; 1.0241x over previous
import functools

import jax
import jax.numpy as jnp
from jax import lax
from jax.experimental import pallas as pl
from jax.experimental.pallas import tpu as pltpu
from jax.experimental.pallas import tpu_sc as plsc

_N_ENT = 50000
_N_USERS = 1024
_E = 800000
_C = 64
_N_REL = 16
_HOPS = 2

_HALF = _C // 2
_CHUNK = 128
_NSUB = 16
_GRP = 6
_NCHUNKP = 6336
_EPAD = _NCHUNKP * _CHUNK
_TCH = _NCHUNKP // _NSUB
_NBLK = _TCH // _GRP
_DE = 4
_DG = 2
_WCHUNK = 400
_NWCHUNK = _N_ENT // _WCHUNK

_LANES = 16

_mesh = plsc.VectorSubcoreMesh(core_axis_name="c", subcore_axis_name="s")


@functools.partial(
    pl.kernel,
    out_type=jax.ShapeDtypeStruct((2, _N_ENT, _HALF), jnp.float32),
    mesh=_mesh,
    scratch_types=[
        pltpu.VMEM((_GRP, 4, _CHUNK), jnp.int32),
        pltpu.VMEM((_GRP, _CHUNK, _HALF), jnp.float32),
        pltpu.VMEM((_N_REL, _HALF), jnp.float32),
        pltpu.VMEM_SHARED((_N_ENT, _HALF), jnp.float32),
        pltpu.SemaphoreType.DMA((_GRP,)),
        pltpu.SemaphoreType.DMA((_GRP,)),
        pltpu.SemaphoreType.DMA((_GRP,)),
    ],
    compiler_params=pltpu.CompilerParams(use_tc_tiling_on_sc=False,
                                         needs_layout_passes=False),
)
def _sc_aggregate(ent_hbm, edges_hbm, relw_hbm, zeros_hbm, out_hbm,
                  ebuf, rows, relw, acc, sem_e, sem_g, sem_s):
    c = lax.axis_index("c")
    s = lax.axis_index("s")

    pltpu.sync_copy(relw_hbm.at[c], relw)

    @pl.loop(s, _NWCHUNK, step=_NSUB)
    def _zcopy(k):
        pltpu.sync_copy(zeros_hbm, acc.at[pl.ds(k * _WCHUNK, _WCHUNK)])

    plsc.subcore_barrier()

    entc = ent_hbm.at[c]

    def chunk_of(k):
        return s + k * _NSUB

    def fire_e(slot, k):
        pltpu.async_copy(edges_hbm.at[chunk_of(k)], ebuf.at[slot],
                         sem_e.at[slot])

    def wait_e(slot):
        pltpu.make_async_copy(edges_hbm.at[0], ebuf.at[slot],
                              sem_e.at[slot]).wait()

    def fire_g(slot):
        pltpu.async_copy(entc.at[ebuf.at[slot, 0]], rows.at[slot],
                         sem_g.at[slot])

    def wait_g(slot):
        pltpu.make_async_copy(entc.at[ebuf.at[slot, 0]], rows.at[slot],
                              sem_g.at[slot]).wait()

    def fire_s(slot):
        pltpu.async_copy(rows.at[slot], acc.at[ebuf.at[slot, 1]],
                         sem_s.at[slot], add=True)

    def wait_s(slot):
        pltpu.make_async_copy(rows.at[slot], acc.at[ebuf.at[slot, 1]],
                              sem_s.at[slot]).wait()

    def compute(slot):
        @pl.loop(0, _CHUNK // _LANES)
        def _sub(q):
            tvec = ebuf[slot, 2, pl.ds(q * _LANES, _LANES)]
            uvec = plsc.bitcast(ebuf[slot, 3, pl.ds(q * _LANES, _LANES)],
                                jnp.float32)
            for ii in range(_LANES):
                i = q * _LANES + ii
                t = tvec[ii]
                u = uvec[ii]
                r0 = relw[t, pl.ds(0, _LANES)] * u
                r1 = relw[t, pl.ds(_LANES, _LANES)] * u
                rows[slot, i, pl.ds(0, _LANES)] = (
                    rows[slot, i, pl.ds(0, _LANES)] * r0)
                rows[slot, i, pl.ds(_LANES, _LANES)] = (
                    rows[slot, i, pl.ds(_LANES, _LANES)] * r1)

    for k0 in range(_DE):
        fire_e(k0, k0)
    for k0 in range(_DG):
        wait_e(k0)
        fire_g(k0)

    @pl.loop(0, _NBLK)
    def _block(bi):
        for q in range(_GRP):
            k = bi * _GRP + q
            wait_g(q)

            r_e = (q + _DE) % _GRP

            @pl.when(jnp.logical_and(k >= _GRP - _DE, k < _TCH - _DE))
            def _():
                wait_s(r_e)
                fire_e(r_e, k + _DE)

            @pl.when(k < _GRP - _DE)
            def _():
                fire_e(r_e, k + _DE)

            r_g = (q + _DG) % _GRP

            @pl.when(k < _TCH - _DG)
            def _():
                wait_e(r_g)
                fire_g(r_g)

            compute(q)
            fire_s(q)

    for q in range(_GRP):
        wait_s(q)

    plsc.subcore_barrier()

    outc = out_hbm.at[c]

    @pl.loop(s, _NWCHUNK, step=_NSUB)
    def _wb(k):
        off = k * _WCHUNK
        pltpu.sync_copy(acc.at[pl.ds(off, _WCHUNK)],
                        outc.at[pl.ds(off, _WCHUNK)])


_UBLK = 64
_USTEPS = _N_USERS // _UBLK


def _mm_body(im_ref, ent_ref, resin_ref, out_ref):
    a = jnp.dot(im_ref[...], ent_ref[...], preferred_element_type=jnp.float32)
    n = jnp.sqrt(jnp.sum(a * a, axis=1, keepdims=True))
    out_ref[...] = resin_ref[...] + a / jnp.maximum(n, 1e-12)


_user_hop = pl.pallas_call(
    _mm_body,
    grid=(_USTEPS,),
    in_specs=[
        pl.BlockSpec((_UBLK, _N_ENT), lambda u: (u, 0)),
        pl.BlockSpec((_N_ENT, _C), lambda u: (0, 0)),
        pl.BlockSpec((_UBLK, _C), lambda u: (u, 0)),
    ],
    out_specs=pl.BlockSpec((_UBLK, _C), lambda u: (u, 0)),
    out_shape=jax.ShapeDtypeStruct((_N_USERS, _C), jnp.float32),
)


_RBLK = 2000


def _norm_body(agg_ref, resin_ref, resout_ref, full_ref, split_ref):
    h0 = agg_ref[0]
    h1 = agg_ref[1]
    n2 = (jnp.sum(h0 * h0, axis=1, keepdims=True)
          + jnp.sum(h1 * h1, axis=1, keepdims=True))
    inv = 1.0 / jnp.maximum(jnp.sqrt(n2), 1e-12)
    e0 = h0 * inv
    e1 = h1 * inv
    full = jnp.concatenate([e0, e1], axis=1)
    resout_ref[...] = resin_ref[...] + full
    full_ref[...] = full
    split_ref[0] = e0
    split_ref[1] = e1


_ent_norm = pl.pallas_call(
    _norm_body,
    grid=(_N_ENT // _RBLK,),
    in_specs=[
        pl.BlockSpec((2, _RBLK, _HALF), lambda i: (0, i, 0)),
        pl.BlockSpec((_RBLK, _C), lambda i: (i, 0)),
    ],
    out_specs=[
        pl.BlockSpec((_RBLK, _C), lambda i: (i, 0)),
        pl.BlockSpec((_RBLK, _C), lambda i: (i, 0)),
        pl.BlockSpec((2, _RBLK, _HALF), lambda i: (0, i, 0)),
    ],
    out_shape=[
        jax.ShapeDtypeStruct((_N_ENT, _C), jnp.float32),
        jax.ShapeDtypeStruct((_N_ENT, _C), jnp.float32),
        jax.ShapeDtypeStruct((2, _N_ENT, _HALF), jnp.float32),
    ],
)


def kernel(user_emb, entity_emb, entity_2nd_emb, user_2nd_emb,
           edge_index, edge_type, interact_mat, weight, triplet_mask):
    del entity_2nd_emb, user_2nd_emb

    ei = edge_index.astype(jnp.int32)
    pad = _EPAD - _E
    tail = jnp.pad(ei[1], (0, pad)).reshape(_NCHUNKP, _CHUNK)
    head = jnp.pad(ei[0], (0, pad)).reshape(_NCHUNKP, _CHUNK)
    et = jnp.pad(edge_type.astype(jnp.int32), (0, pad)).reshape(_NCHUNKP,
                                                                _CHUNK)
    umbits = lax.bitcast_convert_type(
        jnp.pad(triplet_mask, (0, pad)), jnp.int32).reshape(_NCHUNKP, _CHUNK)
    edges_packed = jnp.stack([tail, head, et, umbits], axis=1)

    w2 = jnp.roll(weight, 1, axis=0)
    relw_split = jnp.stack([w2[:, :_HALF], w2[:, _HALF:]], axis=0)
    zblock = jnp.zeros((_WCHUNK, _HALF), jnp.float32)

    ent_full = entity_emb
    ent_split = jnp.stack([entity_emb[:, :_HALF], entity_emb[:, _HALF:]],
                          axis=0)
    ent_res = entity_emb
    usr_res = user_emb

    for _ in range(_HOPS):
        agg = _sc_aggregate(ent_split, edges_packed, relw_split, zblock)
        usr_res = _user_hop(interact_mat, ent_full, usr_res)
        ent_res, ent_full, ent_split = _ent_norm(agg, ent_res)

    return ent_res, usr_res, triplet_mask
